# Initial kernel scaffold; baseline (speedup 1.0000x reference)
import functools
import jax
import jax.numpy as jnp
from jax import lax
from jax.experimental import pallas as pl
from jax.experimental.pallas import tpu as pltpu
from jax.experimental.pallas import tpu_sc as plsc

VOCAB = 1000000
D = 128
B = 8192
L = 16
NC = 2
NS = 16
NW = NC * NS
BPW = B // NW
GCHUNK = 128
NG = BPW // GCHUNK


def _body(idx_hbm, tok_hbm, pos_hbm, out_hbm, idx_v, rows_v, pos_v, sem):
    wid = lax.axis_index("s") * NC + lax.axis_index("c")
    base = wid * BPW

    pltpu.sync_copy(idx_hbm.at[pl.ds(base, BPW)], idx_v)

    copies = []
    for g in range(NG):
        copies.append(
            pltpu.async_copy(
                tok_hbm.at[idx_v.at[g]],
                rows_v.at[pl.ds(g * GCHUNK, GCHUNK), :],
                sem,
            )
        )
    pltpu.sync_copy(pos_hbm.at[pl.ds(base, BPW), :], pos_v)
    for c in copies:
        c.wait()

    def row_add(i, carry):
        for c in range(D // L):
            sl = pl.ds(c * L, L)
            rows_v[i, sl] = rows_v[i, sl] + pos_v[i, sl]
        return carry

    lax.fori_loop(0, BPW, row_add, 0, unroll=2)

    pltpu.sync_copy(rows_v, out_hbm.at[pl.ds(base, BPW), :])


@jax.jit
def _embed(index, tok_table, pos_table):
    mesh = plsc.VectorSubcoreMesh(core_axis_name="c", subcore_axis_name="s")
    kfn = pl.kernel(
        _body,
        out_type=jax.ShapeDtypeStruct((B, D), jnp.float32),
        mesh=mesh,
        scratch_types=[
            pltpu.VMEM((NG, GCHUNK), jnp.int32),
            pltpu.VMEM((BPW, D), jnp.float32),
            pltpu.VMEM((BPW, D), jnp.float32),
            pltpu.SemaphoreType.DMA,
        ],
    )
    return kfn(index, tok_table, pos_table)


def kernel(index, tok_table, pos_table):
    return _embed(index.astype(jnp.int32), tok_table, pos_table)


# speedup vs baseline: 1.2072x; 1.2072x over previous
import functools
import jax
import jax.numpy as jnp
from jax import lax
from jax.experimental import pallas as pl
from jax.experimental.pallas import tpu as pltpu
from jax.experimental.pallas import tpu_sc as plsc

VOCAB = 1000000
D = 128
B = 8192
L = 16
NC = 2
NS = 16
NW = NC * NS
BPW = B // NW
GCHUNK = 128
NG = BPW // GCHUNK


def _body(idx_hbm, tok_hbm, pos_hbm, out_hbm, idx_v, rows_v, pos_v, sem):
    wid = lax.axis_index("s") * NC + lax.axis_index("c")
    base = wid * BPW

    for g in range(NG):
        pltpu.sync_copy(idx_hbm.at[pl.ds(base + g * GCHUNK, GCHUNK)], idx_v.at[g])

    copies = []
    for g in range(NG):
        copies.append(
            pltpu.async_copy(
                tok_hbm.at[idx_v.at[g]],
                rows_v.at[pl.ds(g * GCHUNK, GCHUNK), :],
                sem,
            )
        )
    pltpu.sync_copy(pos_hbm.at[pl.ds(base, BPW), :], pos_v)
    for c in copies:
        c.wait()

    def row_add(i, carry):
        for c in range(D // L):
            sl = pl.ds(c * L, L)
            rows_v[i, sl] = rows_v[i, sl] + pos_v[i, sl]
        return carry

    lax.fori_loop(0, BPW, row_add, 0, unroll=2)

    pltpu.sync_copy(rows_v, out_hbm.at[pl.ds(base, BPW), :])


@jax.jit
def _embed(index, tok_table, pos_table):
    mesh = plsc.VectorSubcoreMesh(core_axis_name="c", subcore_axis_name="s")
    kfn = pl.kernel(
        _body,
        out_type=jax.ShapeDtypeStruct((B, D), jnp.float32),
        mesh=mesh,
        scratch_types=[
            pltpu.VMEM((NG, GCHUNK), jnp.int32),
            pltpu.VMEM((BPW, D), jnp.float32),
            pltpu.VMEM((BPW, D), jnp.float32),
            pltpu.SemaphoreType.DMA,
        ],
    )
    return kfn(index, tok_table, pos_table)


def kernel(index, tok_table, pos_table):
    return _embed(index.astype(jnp.int32), tok_table, pos_table)


# speedup vs baseline: 1.6393x; 1.3579x over previous
import jax
import jax.numpy as jnp
from jax import lax
from jax.experimental import pallas as pl
from jax.experimental.pallas import tpu as pltpu
from jax.experimental.pallas import tpu_sc as plsc

VOCAB = 1000000
D = 128
B = 8192
NC = 2
NS = 16
NW = NC * NS
BPW = B // NW
GCHUNK = 128
NG = BPW // GCHUNK


def _body(idx_hbm, tok_hbm, pos_hbm, out_hbm, idx_v, rows_v, sem):
    wid = lax.axis_index("s") * NC + lax.axis_index("c")
    base = wid * BPW

    for g in range(NG):
        pltpu.sync_copy(idx_hbm.at[pl.ds(base + g * GCHUNK, GCHUNK)], idx_v.at[g])
    pltpu.sync_copy(pos_hbm.at[pl.ds(base, BPW), :], rows_v)
    copies = []
    for g in range(NG):
        copies.append(
            pltpu.async_copy(
                tok_hbm.at[idx_v.at[g]],
                rows_v.at[pl.ds(g * GCHUNK, GCHUNK), :],
                sem,
                add=True,
            )
        )
    for c in copies:
        c.wait()
    pltpu.sync_copy(rows_v, out_hbm.at[pl.ds(base, BPW), :])


@jax.jit
def kernel(index, tok_table, pos_table):
    mesh = plsc.VectorSubcoreMesh(core_axis_name="c", subcore_axis_name="s")
    kfn = pl.kernel(
        _body,
        out_type=jax.ShapeDtypeStruct((B, D), jnp.float32),
        mesh=mesh,
        scratch_types=[
            pltpu.VMEM((NG, GCHUNK), jnp.int32),
            pltpu.VMEM((BPW, D), jnp.float32),
            pltpu.SemaphoreType.DMA,
        ],
    )
    return kfn(index.astype(jnp.int32), tok_table, pos_table)


# speedup vs baseline: 1.6513x; 1.0073x over previous
import jax
import jax.numpy as jnp
from jax import lax
from jax.experimental import pallas as pl
from jax.experimental.pallas import tpu as pltpu
from jax.experimental.pallas import tpu_sc as plsc

VOCAB = 1000000
D = 128
B = 8192
NC = 2
NS = 16
NW = NC * NS
BPW = B // NW
NCH = 4
GC = BPW // NCH


def _body(idx_hbm, tok_hbm, pos_hbm, out_hbm, idx_v, rows_v,
          sem_i, sem_p0, sem_p1, sem_p2, sem_p3,
          sem_g0, sem_g1, sem_g2, sem_g3, sem_o):
    sem_p = [sem_p0, sem_p1, sem_p2, sem_p3]
    sem_g = [sem_g0, sem_g1, sem_g2, sem_g3]
    wid = lax.axis_index("s") * NC + lax.axis_index("c")
    base = wid * BPW

    idx_c, pos_c = [], []
    for k in range(NCH):
        idx_c.append(
            pltpu.async_copy(idx_hbm.at[pl.ds(base + k * GC, GC)], idx_v.at[k], sem_i)
        )
        pos_c.append(
            pltpu.async_copy(
                pos_hbm.at[pl.ds(base + k * GC, GC), :],
                rows_v.at[pl.ds(k * GC, GC), :],
                sem_p[k],
            )
        )
    for c in idx_c:
        c.wait()

    g_c = []
    for k in range(NCH):
        pos_c[k].wait()
        g_c.append(
            pltpu.async_copy(
                tok_hbm.at[idx_v.at[k]],
                rows_v.at[pl.ds(k * GC, GC), :],
                sem_g[k],
                add=True,
            )
        )
    o_c = []
    for k in range(NCH):
        g_c[k].wait()
        o_c.append(
            pltpu.async_copy(
                rows_v.at[pl.ds(k * GC, GC), :],
                out_hbm.at[pl.ds(base + k * GC, GC), :],
                sem_o,
            )
        )
    for c in o_c:
        c.wait()


@jax.jit
def _embed(index, tok_table, pos_table):
    mesh = plsc.VectorSubcoreMesh(core_axis_name="c", subcore_axis_name="s")
    kfn = pl.kernel(
        _body,
        out_type=jax.ShapeDtypeStruct((B, D), jnp.float32),
        mesh=mesh,
        scratch_types=[
            pltpu.VMEM((NCH, GC), jnp.int32),
            pltpu.VMEM((BPW, D), jnp.float32),
        ] + [pltpu.SemaphoreType.DMA] * 10,
    )
    return kfn(index, tok_table, pos_table)


def kernel(index, tok_table, pos_table):
    return _embed(index.astype(jnp.int32), tok_table, pos_table)
